# baseline (speedup 1.0000x reference)
import dataclasses
import functools

import jax
import jax.numpy as jnp
from jax import lax
from jax.experimental import pallas as pl
from jax.experimental.pallas import tpu as pltpu
from jax.experimental.pallas import tpu_sc as plsc

_N = 10000
_E = 320000
_D = 128
_G = 512
_EPS = 1e-5

_NC = 2
_NS = 16
_NW = _NC * _NS
_CHUNK = 80
_CHUNKS_PER_TILE = _E // (_NW * _CHUNK)
_NPAD = 10240
_STRIPE = _NPAD // _NS
_ZROWS = _CHUNK

_OWN = _NPAD // _NW
_HALF = _NPAD // _NC
_ACC_ROWS = _HALF + 8
_CAP2 = 1280
_CAPO = 16640
_EPW = _E // _NW


def _sc_params():
    cp = pltpu.CompilerParams()
    if "needs_layout_passes" in pltpu.CompilerParams.__dataclass_fields__:
        cp = dataclasses.replace(cp, needs_layout_passes=False)
    return cp


def _distribute_sc(epk):
    mesh = plsc.VectorSubcoreMesh(core_axis_name="c", subcore_axis_name="s")

    @functools.partial(
        pl.kernel, mesh=mesh, compiler_params=_sc_params(),
        out_type=[jax.ShapeDtypeStruct((32 * _NW * _CAP2,), jnp.int32),
                  jax.ShapeDtypeStruct((_NW * 32,), jnp.int32)],
        scratch_types=[
            pltpu.VMEM((_CHUNKS_PER_TILE, _CHUNK), jnp.int32),
            pltpu.VMEM((32, _CAP2), jnp.int32),
            pltpu.VMEM((32,), jnp.int32),
        ],
    )
    def ka(epk_hbm, lists_hbm, counts_hbm, win_v, lb_v, cnt_v):
        c = lax.axis_index("c")
        s = lax.axis_index("s")
        wid = s * _NC + c

        cnt_v[pl.ds(0, 16)] = jnp.zeros((16,), jnp.int32)
        cnt_v[pl.ds(16, 16)] = jnp.zeros((16,), jnp.int32)

        pltpu.sync_copy(epk_hbm.at[wid], win_v)

        ones = jnp.ones((16,), jnp.int32)

        @pl.loop(0, _CHUNKS_PER_TILE)
        def _(r):
            @pl.loop(0, _CHUNK // 16)
            def _(k):
                p = win_v[r, pl.ds(k * 16, 16)]
                b = ((p >> 14) * 6554) >> 21
                r0, _unused = plsc.scan_count(b)
                cb = plsc.load_gather(cnt_v, [b])
                pos = jnp.minimum(cb + r0 - 1, _CAP2 - 1)
                plsc.store_scatter(lb_v, [b, pos], p)
                plsc.addupdate_scatter(cnt_v, [b], ones)

        for b in range(32):
            pltpu.sync_copy(lb_v.at[b],
                            lists_hbm.at[pl.ds((b * _NW + wid) * _CAP2, _CAP2)])
        pltpu.sync_copy(cnt_v, counts_hbm.at[pl.ds(wid * 32, 32)])

    return ka(epk)


def _compact_sc(lists, counts):
    mesh = plsc.VectorSubcoreMesh(core_axis_name="c", subcore_axis_name="s")

    @functools.partial(
        pl.kernel, mesh=mesh, compiler_params=_sc_params(),
        out_type=[jax.ShapeDtypeStruct((_NW * _CAPO,), jnp.int32),
                  jax.ShapeDtypeStruct((_NW * 16,), jnp.int32)],
        scratch_types=[
            pltpu.VMEM((2, _CAP2), jnp.int32),
            pltpu.VMEM((32 * 32,), jnp.int32),
            pltpu.VMEM((_CAPO,), jnp.int32),
            pltpu.VMEM((16,), jnp.int32),
            pltpu.SemaphoreType.DMA,
            pltpu.SemaphoreType.DMA,
        ],
    )
    def kb(lists_hbm, counts_hbm, olists_hbm, ocnt_hbm,
           stage_v, cnts_v, big_v, nst_v, sem0, sem1):
        c = lax.axis_index("c")
        s = lax.axis_index("s")
        o = c * 16 + s

        pltpu.sync_copy(counts_hbm, cnts_v)

        iota = lax.iota(jnp.int32, 16)
        sems = (sem0, sem1)

        def sub_load(k, bb):
            return pltpu.make_async_copy(
                lists_hbm.at[pl.ds((o * _NW + k) * _CAP2, _CAP2)],
                stage_v.at[bb], sems[bb])

        def cnt_of(k):
            idx = k * 32 + o
            base = (idx // 16) * 16
            off = idx % 16
            v = cnts_v[pl.ds(base, 16)]
            return jnp.sum(jnp.where(iota == off, v, 0))

        sub_load(0, 0).start()
        fill = jnp.int32(0)
        for k in range(32):
            bb = k % 2
            sub_load(k, bb).wait()
            if k + 1 < 32:
                sub_load(k + 1, 1 - bb).start()
            cnt_k = jnp.minimum(cnt_of(k), _CAP2)
            nv = (cnt_k + 15) >> 4
            fill_k = fill

            @pl.loop(0, nv)
            def _(i):
                vals = stage_v[bb, pl.ds(i * 16, 16)]
                m = iota < (cnt_k - i * 16)
                idx = jnp.minimum(fill_k + i * 16 + iota, _CAPO - 1)
                plsc.store_scatter(big_v, [idx], vals, mask=m)

            fill = jnp.minimum(fill + cnt_k, _CAPO - 241)

        trash = jnp.full((16,), (c * _HALF + _HALF) << 14, jnp.int32)
        for i in range(15):
            plsc.store_scatter(big_v, [fill + i * 16 + iota], trash)

        nchunks = jnp.maximum((((fill + 79) // 80) + 1) & -2, 2)
        nst_v[...] = jnp.full((16,), 0, jnp.int32) + nchunks
        pltpu.sync_copy(big_v, olists_hbm.at[pl.ds(o * _CAPO, _CAPO)])
        pltpu.sync_copy(nst_v, ocnt_hbm.at[pl.ds(o * 16, 16)])

    return kb(lists, counts)


def _segsum_ordered_sc(h, olists, ocnt):
    mesh = plsc.VectorSubcoreMesh(core_axis_name="c", subcore_axis_name="s")

    @functools.partial(
        pl.kernel,
        mesh=mesh,
        compiler_params=_sc_params(),
        out_type=jax.ShapeDtypeStruct((_NPAD, _D), jnp.float32),
        scratch_types=[
            pltpu.VMEM((_CAPO,), jnp.int32),
            pltpu.VMEM((2, _CHUNK), jnp.int32),
            pltpu.VMEM((2, _CHUNK), jnp.int32),
            pltpu.VMEM((2, _CHUNK, _D), jnp.float32),
            pltpu.VMEM((16,), jnp.int32),
            pltpu.VMEM_SHARED((_ACC_ROWS, _D), jnp.float32),
            pltpu.SemaphoreType.DMA,
            pltpu.SemaphoreType.DMA,
        ],
    )
    def kc(h_hbm, ol_hbm, oc_hbm, out_hbm, eidx_v, srcu, dstu, rows_v, nck_v,
           acc, gsem0, gsem1):
        c = lax.axis_index("c")
        s = lax.axis_index("s")
        o = c * 16 + s

        @pl.loop(0, _ZROWS)
        def _(i):
            @pl.loop(0, _D // 16)
            def _(j):
                rows_v[0, i, pl.ds(j * 16, 16)] = jnp.zeros((16,), jnp.float32)

        lbase = s * _OWN

        @pl.loop(0, _OWN // _ZROWS)
        def _(r):
            pltpu.sync_copy(rows_v.at[0], acc.at[pl.ds(lbase + r * _ZROWS, _ZROWS)])

        pltpu.sync_copy(ol_hbm.at[pl.ds(o * _CAPO, _CAPO)], eidx_v)
        pltpu.sync_copy(oc_hbm.at[pl.ds(o * 16, 16)], nck_v)
        nchunks = jnp.max(nck_v[...])

        sems = (gsem0, gsem1)

        def unpack(j, b):
            @pl.loop(0, _CHUNK // 16)
            def _(k):
                p = eidx_v[pl.ds(j * _CHUNK + k * 16, 16)]
                srcu[b, pl.ds(k * 16, 16)] = p & 0x3FFF
                dstu[b, pl.ds(k * 16, 16)] = (p >> 14) - c * _HALF

        def g_copy(b):
            return pltpu.make_async_copy(
                h_hbm.at[srcu.at[b]], rows_v.at[b], sems[b])

        def s_sync(b):
            pltpu.sync_copy(rows_v.at[b], acc.at[dstu.at[b]], add=True)

        unpack(0, 0)
        g_copy(0).start()

        @pl.loop(0, nchunks // 2)
        def _(i):
            j0 = 2 * i
            unpack(j0 + 1, 1)
            g_copy(1).start()
            g_copy(0).wait()
            s_sync(0)

            @pl.when(j0 + 2 < nchunks)
            def _():
                unpack(j0 + 2, 0)
                g_copy(0).start()

            g_copy(1).wait()
            s_sync(1)

        pltpu.sync_copy(acc.at[pl.ds(lbase, _OWN)],
                        out_hbm.at[pl.ds(o * _OWN, _OWN)])

    return kc(h, olists, ocnt)


def _gin_layer_tc(h, agg, w1, b1, w2, b2, gamma, beta, relu_out):
    def body(h_ref, a_ref, w1_ref, b1_ref, w2_ref, b2_ref, g_ref, be_ref, o_ref):
        t = h_ref[...] + a_ref[:_N]
        u = jnp.dot(t, w1_ref[...], preferred_element_type=jnp.float32) + b1_ref[...]
        u = jnp.maximum(u, 0.0)
        v = jnp.dot(u, w2_ref[...], preferred_element_type=jnp.float32) + b2_ref[...]
        mean = jnp.mean(v, axis=0, keepdims=True)
        cen = v - mean
        var = jnp.mean(cen * cen, axis=0, keepdims=True)
        o = cen * (g_ref[...] * lax.rsqrt(var + _EPS)) + be_ref[...]
        if relu_out:
            o = jnp.maximum(o, 0.0)
        o_ref[...] = o

    return pl.pallas_call(
        body, out_shape=jax.ShapeDtypeStruct((_N, _D), jnp.float32)
    )(h, agg, w1, b1, w2, b2, gamma, beta)


def _pool_linear_tc(h, batch2d, lin_w, lin_b):
    def body(h_ref, b_ref, w_ref, bias_ref, o_ref):
        ids = lax.broadcasted_iota(jnp.int32, (_N, _G), 1)
        oh = (b_ref[...] == ids).astype(jnp.float32)
        sums = lax.dot_general(oh, h_ref[...], (((0,), (0,)), ((), ())),
                               preferred_element_type=jnp.float32,
                               precision=lax.Precision.HIGHEST)
        cnt = jnp.sum(oh, axis=0)[:, None]
        pooled = sums / jnp.maximum(cnt, 1.0)
        o_ref[...] = jnp.dot(pooled, w_ref[...],
                             preferred_element_type=jnp.float32) + bias_ref[...]

    return pl.pallas_call(
        body, out_shape=jax.ShapeDtypeStruct((_G, _D), jnp.float32)
    )(h, batch2d, lin_w, lin_b)


def kernel(x, edge_index, batch,
           w1_0, b1_0, w2_0, b2_0, gamma_0, beta_0,
           w1_1, b1_1, w2_1, b2_1, gamma_1, beta_1,
           w1_2, b1_2, w2_2, b2_2, gamma_2, beta_2,
           lin_w, lin_b):
    epk = (edge_index[0] | (edge_index[1] << 14)).reshape(
        _NW, _CHUNKS_PER_TILE, _CHUNK)
    batch2d = batch.reshape(_N, 1)
    lists, counts = _distribute_sc(epk)
    olists, ocnt = _compact_sc(lists, counts)

    layers = [
        (w1_0, b1_0, w2_0, b2_0, gamma_0, beta_0),
        (w1_1, b1_1, w2_1, b2_1, gamma_1, beta_1),
        (w1_2, b1_2, w2_2, b2_2, gamma_2, beta_2),
    ]
    h = x
    for i, (w1, b1, w2, b2, g, be) in enumerate(layers):
        agg = _segsum_ordered_sc(h, olists, ocnt)
        h = _gin_layer_tc(h, agg, w1, b1.reshape(1, _D), w2, b2.reshape(1, _D),
                          g.reshape(1, _D), be.reshape(1, _D), relu_out=(i < 2))
    return _pool_linear_tc(h, batch2d, lin_w, lin_b)
